# baseline (speedup 1.0000x reference)
import functools

import jax
import jax.numpy as jnp
from jax import lax
from jax.experimental import pallas as pl
from jax.experimental.pallas import tpu as pltpu
from jax.experimental.pallas import tpu_sc as plsc

N = 10000
E = 320000
D_IN = 128
D_HID = 128
D_OUT = 64

NC = 2
NS = 16
NW = NC * NS

PKW = 128
EPW = 10240
EP = NW * EPW
NPK = EPW // PKW
NPAD = 10240
RPT = NPAD // NS
DUMMY = N


def _sc_mesh():
    return plsc.VectorSubcoreMesh(core_axis_name="c", subcore_axis_name="s",
                                  num_cores=NC, num_subcores=NS)


_DNB = 8


def _deg_body(pk_hbm, out_hbm, pkvm, rowv, ones_v, stage, acc, *sems):
    c = lax.axis_index("c")
    s = lax.axis_index("s")
    wid = s * NC + c
    for k in range(PKW // 16):
        ones_v[pl.ds(k * 16, 16)] = jnp.ones((16,), jnp.float32)

    def zloop(i, carry):
        stage[pl.ds(i * 16, 16)] = jnp.zeros((16,), jnp.float32)
        return carry

    lax.fori_loop(0, RPT // 16, zloop, 0)
    pltpu.sync_copy(pk_hbm.at[pl.ds(wid * EPW, EPW)], pkvm)
    pltpu.sync_copy(stage, acc.at[pl.ds(s * RPT, RPT)])
    plsc.subcore_barrier()

    def unpack(m, b):
        for k in range(PKW // 16):
            v = pkvm[pl.ds(m * PKW + k * 16, 16)]
            rowv[b][pl.ds(k * 16, 16)] = lax.shift_right_logical(v, 16)

    def fire(b):
        return pltpu.async_copy(ones_v, acc.at[rowv[b]], sems[b], add=True)

    def drain(b):
        pltpu.make_async_copy(ones_v, acc.at[rowv[b]], sems[b]).wait()

    for b in range(_DNB):
        unpack(b, b)
        fire(b)

    def group(g, carry):
        for b in range(_DNB):
            drain(b)
            unpack(_DNB * g + b, b)
            fire(b)
        return carry

    lax.fori_loop(1, NPK // _DNB, group, 0)
    for b in range(_DNB):
        drain(b)
    plsc.subcore_barrier()
    pltpu.sync_copy(acc.at[pl.ds(s * RPT, RPT)], stage)
    pltpu.sync_copy(stage, out_hbm.at[c, pl.ds(s * RPT, RPT)])


@functools.cache
def _deg_call():
    return pl.kernel(
        _deg_body,
        out_type=jax.ShapeDtypeStruct((NC, NPAD), jnp.float32),
        mesh=_sc_mesh(),
        scratch_types=[
            pltpu.VMEM((EPW,), jnp.int32),
            [pltpu.VMEM((PKW,), jnp.int32) for _ in range(_DNB)],
            pltpu.VMEM((PKW,), jnp.float32),
            pltpu.VMEM((RPT,), jnp.float32),
            pltpu.VMEM_SHARED((NPAD,), jnp.float32),
        ] + [pltpu.SemaphoreType.DMA] * _DNB,
        compiler_params=pltpu.CompilerParams(use_tc_tiling_on_sc=False),
    )


_NB = 5
_K = 2


def _spmm_body(d, ck, halves, nb, g_hbm, pk_hbm, out_hbm,
               pkvm, colv, rowv, rows, acc, isem, semg, sems):
    c = lax.axis_index("c")
    s = lax.axis_index("s")
    wid = s * NC + c
    nchunk = EPW // ck
    nz = RPT // ck
    win = EPW // halves

    def zrow(r, carry):
        for k in range(d // 16):
            rows[0][r, pl.ds(k * 16, 16)] = jnp.zeros((16,), jnp.float32)
        return carry

    lax.fori_loop(0, ck, zrow, 0)
    zcp = [pltpu.async_copy(rows[0],
                            acc.at[pl.ds(s * RPT + i * ck, ck)], isem)
           for i in range(nz)]
    pltpu.sync_copy(pk_hbm.at[pl.ds(wid * EPW, win)], pkvm)
    for z in zcp:
        z.wait()
    plsc.subcore_barrier()

    def unpack(j, b):
        off = lax.rem(j * ck, win) if halves > 1 else j * ck
        for k in range(ck // 16):
            v = pkvm[pl.ds(off + k * 16, 16)]
            colv[b][pl.ds(k * 16, 16)] = jnp.bitwise_and(v, 0xFFFF)
            rowv[b][pl.ds(k * 16, 16)] = lax.shift_right_logical(v, 16)

    def fire_g(j, b):
        return pltpu.async_copy(g_hbm.at[colv[b]], rows[b], semg[b])

    def wait_g(j, b):
        pltpu.make_async_copy(g_hbm.at[colv[b]], rows[b], semg[b]).wait()

    def fire_s(j, b):
        return pltpu.async_copy(rows[b], acc.at[rowv[b]], sems[b], add=True)

    def wait_s(j, b):
        pltpu.make_async_copy(rows[b], acc.at[rowv[b]], sems[b]).wait()

    for b in range(nb):
        unpack(b, b)
        fire_g(b, b)
    for j in range(nb - _K):
        wait_g(j, j)
        fire_s(j, j)

    def group(g, carry):
        for b in range(nb):
            j = nb * g + b
            wait_s(j - nb, b)
            unpack(j, b)
            fire_g(j, b)
            jk = j - _K
            bk = (b - _K) % nb
            wait_g(jk, bk)
            fire_s(jk, bk)
        return carry

    ngrp = nchunk // nb
    if halves == 1:
        lax.fori_loop(1, ngrp, group, 0)
    else:
        lax.fori_loop(1, ngrp // 2, group, 0)
        pltpu.sync_copy(pk_hbm.at[pl.ds(wid * EPW + win, win)], pkvm)
        lax.fori_loop(ngrp // 2, ngrp, group, 0)
    for j in range(nchunk - _K, nchunk):
        wait_g(j, j % nb)
        fire_s(j, j % nb)
    for b in range(nb):
        wait_s(nchunk - nb + b, b)
    plsc.subcore_barrier()

    def rd_pair(i):
        return (acc.at[pl.ds(s * RPT + i * ck, ck)], rows[i % nb],
                semg[i % nb])

    def wr_pair(i):
        return (rows[i % nb], out_hbm.at[c, pl.ds(s * RPT + i * ck, ck)],
                sems[i % nb])

    for i in range(nz):
        if i >= nb:
            pltpu.make_async_copy(*wr_pair(i - nb)).wait()
        pltpu.async_copy(*rd_pair(i))
        if i >= 1:
            pltpu.make_async_copy(*rd_pair(i - 1)).wait()
            pltpu.async_copy(*wr_pair(i - 1))
    pltpu.make_async_copy(*rd_pair(nz - 1)).wait()
    pltpu.async_copy(*wr_pair(nz - 1))
    for i in range(max(0, nz - nb), nz):
        pltpu.make_async_copy(*wr_pair(i)).wait()


@functools.cache
def _make_spmm(d, ck, halves, nb):
    return pl.kernel(
        functools.partial(_spmm_body, d, ck, halves, nb),
        out_type=jax.ShapeDtypeStruct((NC, NPAD, d), jnp.float32),
        mesh=_sc_mesh(),
        scratch_types=[
            pltpu.VMEM((EPW // halves,), jnp.int32),
            [pltpu.VMEM((ck,), jnp.int32) for _ in range(nb)],
            [pltpu.VMEM((ck,), jnp.int32) for _ in range(nb)],
            [pltpu.VMEM((ck, d), jnp.float32) for _ in range(nb)],
            pltpu.VMEM_SHARED((NPAD, d), jnp.float32),
            pltpu.SemaphoreType.DMA,
            [pltpu.SemaphoreType.DMA for _ in range(nb)],
            [pltpu.SemaphoreType.DMA for _ in range(nb)],
        ],
        compiler_params=pltpu.CompilerParams(use_tc_tiling_on_sc=False),
    )


_RB = 2048
_GRID = (N + _RB - 1) // _RB


def _pack_body(ei_ref, pk_ref):
    pk_ref[...] = jnp.left_shift(ei_ref[0], 16) | ei_ref[1]


def _pack(ei):
    return pl.pallas_call(
        _pack_body,
        out_shape=jax.ShapeDtypeStruct((E,), jnp.int32),
    )(ei)


def _dense0_body(deg_ref, x_ref, w_ref, dinv_ref, g0_ref):
    dinv = lax.rsqrt(deg_ref[0] + deg_ref[1] + 1.0)
    dinv_ref[...] = dinv
    g0_ref[...] = dinv[:, None] * jnp.dot(x_ref[...], w_ref[...],
                                          preferred_element_type=jnp.float32)


def _dense0(degp, x, w0):
    return pl.pallas_call(
        _dense0_body,
        grid=(_GRID,),
        in_specs=[
            pl.BlockSpec((NC, _RB), lambda i: (0, i)),
            pl.BlockSpec((_RB, D_IN), lambda i: (i, 0)),
            pl.BlockSpec((D_IN, D_HID), lambda i: (0, 0)),
        ],
        out_specs=[
            pl.BlockSpec((_RB,), lambda i: (i,)),
            pl.BlockSpec((_RB, D_HID), lambda i: (i, 0)),
        ],
        out_shape=[
            jax.ShapeDtypeStruct((N,), jnp.float32),
            jax.ShapeDtypeStruct((N, D_HID), jnp.float32),
        ],
    )(degp, x, w0)


def _dense1_body(p_ref, g0_ref, dinv_ref, m1_ref, m2_ref, w_ref, g1_ref):
    dinv = dinv_ref[...][:, None]
    a = dinv * (p_ref[0] + p_ref[1] + g0_ref[...])
    h = jnp.maximum(a, 0.0) * m1_ref[...] * m2_ref[...]
    g1_ref[...] = dinv * jnp.dot(h, w_ref[...],
                                 preferred_element_type=jnp.float32)


def _dense1(acc0, g0, dinv, m1, m2, w1):
    return pl.pallas_call(
        _dense1_body,
        grid=(_GRID,),
        in_specs=[
            pl.BlockSpec((NC, _RB, D_HID), lambda i: (0, i, 0)),
            pl.BlockSpec((_RB, D_HID), lambda i: (i, 0)),
            pl.BlockSpec((_RB,), lambda i: (i,)),
            pl.BlockSpec((_RB, D_HID), lambda i: (i, 0)),
            pl.BlockSpec((_RB, D_HID), lambda i: (i, 0)),
            pl.BlockSpec((D_HID, D_OUT), lambda i: (0, 0)),
        ],
        out_specs=pl.BlockSpec((_RB, D_OUT), lambda i: (i, 0)),
        out_shape=jax.ShapeDtypeStruct((N, D_OUT), jnp.float32),
    )(acc0, g0, dinv, m1, m2, w1)


def _dense2_body(q_ref, g1_ref, dinv_ref, out_ref):
    out_ref[...] = dinv_ref[...][:, None] * (q_ref[0] + q_ref[1]
                                             + g1_ref[...])


def _dense2(acc1, g1, dinv):
    return pl.pallas_call(
        _dense2_body,
        grid=(_GRID,),
        in_specs=[
            pl.BlockSpec((NC, _RB, D_OUT), lambda i: (0, i, 0)),
            pl.BlockSpec((_RB, D_OUT), lambda i: (i, 0)),
            pl.BlockSpec((_RB,), lambda i: (i,)),
        ],
        out_specs=pl.BlockSpec((_RB, D_OUT), lambda i: (i, 0)),
        out_shape=jax.ShapeDtypeStruct((N, D_OUT), jnp.float32),
    )(acc1, g1, dinv)


def kernel(x, edge_index, W0, W1, adj_mask1_train, adj_mask2_fixed):
    ei = edge_index.astype(jnp.int32)
    npad = EP - E
    ar = jnp.arange(npad, dtype=jnp.int32)
    pad = jnp.left_shift(DUMMY + ar % (NPAD - N), 16) | (ar % N)

    packed = jnp.concatenate([_pack(ei), pad])
    degp = _deg_call()(packed)
    dinv, g0 = _dense0(degp, x, W0)
    acc0 = _make_spmm(D_HID, 64, 2, 5)(g0, packed)
    g1 = _dense1(acc0, g0, dinv,
                 adj_mask1_train, adj_mask2_fixed, W1)
    acc1 = _make_spmm(D_OUT, 128, 1, 8)(g1, packed)
    return _dense2(acc1, g1, dinv)


# speedup vs baseline: 1.0176x; 1.0176x over previous
import functools

import jax
import jax.numpy as jnp
from jax import lax
from jax.experimental import pallas as pl
from jax.experimental.pallas import tpu as pltpu
from jax.experimental.pallas import tpu_sc as plsc

N = 10000
E = 320000
D_IN = 128
D_HID = 128
D_OUT = 64

NC = 2
NS = 16
NW = NC * NS

PKW = 128
EPW = 10240
EP = NW * EPW
NPK = EPW // PKW
NPAD = 10240
RPT = NPAD // NS
DUMMY = N


def _sc_mesh():
    return plsc.VectorSubcoreMesh(core_axis_name="c", subcore_axis_name="s",
                                  num_cores=NC, num_subcores=NS)


_DNB = 8


def _deg_body(pk_hbm, out_hbm, pkvm, rowv, ones_v, stage, acc, *sems):
    c = lax.axis_index("c")
    s = lax.axis_index("s")
    wid = s * NC + c
    for k in range(PKW // 16):
        ones_v[pl.ds(k * 16, 16)] = jnp.ones((16,), jnp.float32)

    def zloop(i, carry):
        stage[pl.ds(i * 16, 16)] = jnp.zeros((16,), jnp.float32)
        return carry

    lax.fori_loop(0, RPT // 16, zloop, 0)
    pltpu.sync_copy(pk_hbm.at[pl.ds(wid * EPW, EPW)], pkvm)
    pltpu.sync_copy(stage, acc.at[pl.ds(s * RPT, RPT)])
    plsc.subcore_barrier()

    def unpack(m, b):
        for k in range(PKW // 16):
            v = pkvm[pl.ds(m * PKW + k * 16, 16)]
            rowv[b][pl.ds(k * 16, 16)] = lax.shift_right_logical(v, 16)

    def fire(b):
        return pltpu.async_copy(ones_v, acc.at[rowv[b]], sems[b], add=True)

    def drain(b):
        pltpu.make_async_copy(ones_v, acc.at[rowv[b]], sems[b]).wait()

    for b in range(_DNB):
        unpack(b, b)
        fire(b)

    def group(g, carry):
        for b in range(_DNB):
            drain(b)
            unpack(_DNB * g + b, b)
            fire(b)
        return carry

    lax.fori_loop(1, NPK // _DNB, group, 0)
    for b in range(_DNB):
        drain(b)
    plsc.subcore_barrier()
    pltpu.sync_copy(acc.at[pl.ds(s * RPT, RPT)], stage)
    pltpu.sync_copy(stage, out_hbm.at[c, pl.ds(s * RPT, RPT)])


@functools.cache
def _deg_call():
    return pl.kernel(
        _deg_body,
        out_type=jax.ShapeDtypeStruct((NC, NPAD), jnp.float32),
        mesh=_sc_mesh(),
        scratch_types=[
            pltpu.VMEM((EPW,), jnp.int32),
            [pltpu.VMEM((PKW,), jnp.int32) for _ in range(_DNB)],
            pltpu.VMEM((PKW,), jnp.float32),
            pltpu.VMEM((RPT,), jnp.float32),
            pltpu.VMEM_SHARED((NPAD,), jnp.float32),
        ] + [pltpu.SemaphoreType.DMA] * _DNB,
        compiler_params=pltpu.CompilerParams(use_tc_tiling_on_sc=False),
    )


_NB = 5
_K = 2


def _spmm_body(d, ck, halves, nb, g_hbm, pk_hbm, out_hbm,
               pkvm, colv, rowv, rows, acc, isem, semg, sems):
    c = lax.axis_index("c")
    s = lax.axis_index("s")
    wid = s * NC + c
    nchunk = EPW // ck
    nz = RPT // ck
    win = EPW // halves

    def zrow(r, carry):
        for k in range(d // 16):
            rows[0][r, pl.ds(k * 16, 16)] = jnp.zeros((16,), jnp.float32)
        return carry

    lax.fori_loop(0, ck, zrow, 0)
    zcp = [pltpu.async_copy(rows[0],
                            acc.at[pl.ds(s * RPT + i * ck, ck)], isem)
           for i in range(nz)]
    pltpu.sync_copy(pk_hbm.at[pl.ds(wid * EPW, win)], pkvm)
    for z in zcp:
        z.wait()
    plsc.subcore_barrier()

    def unpack(j, b):
        off = lax.rem(j * ck, win) if halves > 1 else j * ck
        for k in range(ck // 16):
            v = pkvm[pl.ds(off + k * 16, 16)]
            colv[b][pl.ds(k * 16, 16)] = jnp.bitwise_and(v, 0xFFFF)
            rowv[b][pl.ds(k * 16, 16)] = lax.shift_right_logical(v, 16)

    def fire_g(j, b):
        return pltpu.async_copy(g_hbm.at[colv[b]], rows[b], semg[b])

    def wait_g(j, b):
        pltpu.make_async_copy(g_hbm.at[colv[b]], rows[b], semg[b]).wait()

    def fire_s(j, b):
        return pltpu.async_copy(rows[b], acc.at[rowv[b]], sems[b], add=True)

    def wait_s(j, b):
        pltpu.make_async_copy(rows[b], acc.at[rowv[b]], sems[b]).wait()

    for b in range(nb):
        unpack(b, b)
        fire_g(b, b)
    for j in range(nb - _K):
        wait_g(j, j)
        fire_s(j, j)

    def group(g, carry):
        for b in range(nb):
            j = nb * g + b
            wait_s(j - nb, b)
            unpack(j, b)
            fire_g(j, b)
            jk = j - _K
            bk = (b - _K) % nb
            wait_g(jk, bk)
            fire_s(jk, bk)
        return carry

    ngrp = nchunk // nb
    if halves == 1:
        lax.fori_loop(1, ngrp, group, 0)
    else:
        lax.fori_loop(1, ngrp // 2, group, 0)
        pltpu.sync_copy(pk_hbm.at[pl.ds(wid * EPW + win, win)], pkvm)
        lax.fori_loop(ngrp // 2, ngrp, group, 0)
    for j in range(nchunk - _K, nchunk):
        wait_g(j, j % nb)
        fire_s(j, j % nb)
    for b in range(nb):
        wait_s(nchunk - nb + b, b)
    plsc.subcore_barrier()

    def rd_pair(i):
        return (acc.at[pl.ds(s * RPT + i * ck, ck)], rows[i % nb],
                semg[i % nb])

    def wr_pair(i):
        return (rows[i % nb], out_hbm.at[c, pl.ds(s * RPT + i * ck, ck)],
                sems[i % nb])

    for i in range(nz):
        if i >= nb:
            pltpu.make_async_copy(*wr_pair(i - nb)).wait()
        pltpu.async_copy(*rd_pair(i))
        if i >= 1:
            pltpu.make_async_copy(*rd_pair(i - 1)).wait()
            pltpu.async_copy(*wr_pair(i - 1))
    pltpu.make_async_copy(*rd_pair(nz - 1)).wait()
    pltpu.async_copy(*wr_pair(nz - 1))
    for i in range(max(0, nz - nb), nz):
        pltpu.make_async_copy(*wr_pair(i)).wait()


@functools.cache
def _make_spmm(d, ck, halves, nb):
    return pl.kernel(
        functools.partial(_spmm_body, d, ck, halves, nb),
        out_type=jax.ShapeDtypeStruct((NC, NPAD, d), jnp.float32),
        mesh=_sc_mesh(),
        scratch_types=[
            pltpu.VMEM((EPW // halves,), jnp.int32),
            [pltpu.VMEM((ck,), jnp.int32) for _ in range(nb)],
            [pltpu.VMEM((ck,), jnp.int32) for _ in range(nb)],
            [pltpu.VMEM((ck, d), jnp.float32) for _ in range(nb)],
            pltpu.VMEM_SHARED((NPAD, d), jnp.float32),
            pltpu.SemaphoreType.DMA,
            [pltpu.SemaphoreType.DMA for _ in range(nb)],
            [pltpu.SemaphoreType.DMA for _ in range(nb)],
        ],
        compiler_params=pltpu.CompilerParams(use_tc_tiling_on_sc=False),
    )


_RB = 2048
_GRID = (N + _RB - 1) // _RB


def _pack_body(ei_ref, pk_ref):
    pk_ref[...] = jnp.left_shift(ei_ref[0], 16) | ei_ref[1]


def _pack(ei):
    return pl.pallas_call(
        _pack_body,
        out_shape=jax.ShapeDtypeStruct((E,), jnp.int32),
    )(ei)


def _dense0_body(deg_ref, x_ref, w_ref, dinv_ref, g0_ref):
    dinv = lax.rsqrt(deg_ref[0] + deg_ref[1] + 1.0)
    dinv_ref[...] = dinv
    g0_ref[...] = dinv[:, None] * jnp.dot(x_ref[...], w_ref[...],
                                          preferred_element_type=jnp.float32)


def _dense0(degp, x, w0):
    return pl.pallas_call(
        _dense0_body,
        grid=(_GRID,),
        in_specs=[
            pl.BlockSpec((NC, _RB), lambda i: (0, i)),
            pl.BlockSpec((_RB, D_IN), lambda i: (i, 0)),
            pl.BlockSpec((D_IN, D_HID), lambda i: (0, 0)),
        ],
        out_specs=[
            pl.BlockSpec((_RB,), lambda i: (i,)),
            pl.BlockSpec((_RB, D_HID), lambda i: (i, 0)),
        ],
        out_shape=[
            jax.ShapeDtypeStruct((N,), jnp.float32),
            jax.ShapeDtypeStruct((N, D_HID), jnp.float32),
        ],
    )(degp, x, w0)


def _dense1_body(p_ref, g0_ref, dinv_ref, m1_ref, m2_ref, w_ref, g1_ref):
    dinv = dinv_ref[...][:, None]
    a = dinv * (p_ref[0] + p_ref[1] + g0_ref[...])
    h = jnp.maximum(a, 0.0) * m1_ref[...] * m2_ref[...]
    g1_ref[...] = dinv * jnp.dot(h, w_ref[...],
                                 preferred_element_type=jnp.float32)


def _dense1(acc0, g0, dinv, m1, m2, w1):
    return pl.pallas_call(
        _dense1_body,
        grid=(_GRID,),
        in_specs=[
            pl.BlockSpec((NC, _RB, D_HID), lambda i: (0, i, 0)),
            pl.BlockSpec((_RB, D_HID), lambda i: (i, 0)),
            pl.BlockSpec((_RB,), lambda i: (i,)),
            pl.BlockSpec((_RB, D_HID), lambda i: (i, 0)),
            pl.BlockSpec((_RB, D_HID), lambda i: (i, 0)),
            pl.BlockSpec((D_HID, D_OUT), lambda i: (0, 0)),
        ],
        out_specs=pl.BlockSpec((_RB, D_OUT), lambda i: (i, 0)),
        out_shape=jax.ShapeDtypeStruct((N, D_OUT), jnp.float32),
    )(acc0, g0, dinv, m1, m2, w1)


def _dense2_body(q_ref, g1_ref, dinv_ref, out_ref):
    out_ref[...] = dinv_ref[...][:, None] * (q_ref[0] + q_ref[1]
                                             + g1_ref[...])


def _dense2(acc1, g1, dinv):
    return pl.pallas_call(
        _dense2_body,
        grid=(_GRID,),
        in_specs=[
            pl.BlockSpec((NC, _RB, D_OUT), lambda i: (0, i, 0)),
            pl.BlockSpec((_RB, D_OUT), lambda i: (i, 0)),
            pl.BlockSpec((_RB,), lambda i: (i,)),
        ],
        out_specs=pl.BlockSpec((_RB, D_OUT), lambda i: (i, 0)),
        out_shape=jax.ShapeDtypeStruct((N, D_OUT), jnp.float32),
    )(acc1, g1, dinv)


def kernel(x, edge_index, W0, W1, adj_mask1_train, adj_mask2_fixed):
    ei = edge_index.astype(jnp.int32)
    npad = EP - E
    ar = jnp.arange(npad, dtype=jnp.int32)
    pad = jnp.left_shift(DUMMY + ar % (NPAD - N), 16) | (ar % N)

    packed = jnp.concatenate([_pack(ei), pad])
    degp = _deg_call()(packed)
    dinv, g0 = _dense0(degp, x, W0)
    acc0 = _make_spmm(D_HID, 64, 2, 5)(g0, packed)
    g1 = _dense1(acc0, g0, dinv,
                 adj_mask1_train, adj_mask2_fixed, W1)
    acc1 = _make_spmm(D_OUT, 128, 1, 5)(g1, packed)
    return _dense2(acc1, g1, dinv)


# speedup vs baseline: 1.0488x; 1.0307x over previous
import functools

import jax
import jax.numpy as jnp
from jax import lax
from jax.experimental import pallas as pl
from jax.experimental.pallas import tpu as pltpu
from jax.experimental.pallas import tpu_sc as plsc

N = 10000
E = 320000
D_IN = 128
D_HID = 128
D_OUT = 64

NC = 2
NS = 16
NW = NC * NS

PKW = 128
EPW = 10240
EP = NW * EPW
NPK = EPW // PKW
NPAD = 10240
RPT = NPAD // NS
DUMMY = N


def _sc_mesh():
    return plsc.VectorSubcoreMesh(core_axis_name="c", subcore_axis_name="s",
                                  num_cores=NC, num_subcores=NS)


_DNB = 8


def _deg_body(pk_hbm, out_hbm, pkvm, rowv, ones_v, stage, acc, *sems):
    c = lax.axis_index("c")
    s = lax.axis_index("s")
    wid = s * NC + c
    for k in range(PKW // 16):
        ones_v[pl.ds(k * 16, 16)] = jnp.ones((16,), jnp.float32)

    def zloop(i, carry):
        stage[pl.ds(i * 16, 16)] = jnp.zeros((16,), jnp.float32)
        return carry

    lax.fori_loop(0, RPT // 16, zloop, 0)
    pltpu.sync_copy(pk_hbm.at[pl.ds(wid * EPW, EPW)], pkvm)
    pltpu.sync_copy(stage, acc.at[pl.ds(s * RPT, RPT)])
    plsc.subcore_barrier()

    def unpack(m, b):
        for k in range(PKW // 16):
            v = pkvm[pl.ds(m * PKW + k * 16, 16)]
            rowv[b][pl.ds(k * 16, 16)] = lax.shift_right_logical(v, 16)

    def fire(b):
        return pltpu.async_copy(ones_v, acc.at[rowv[b]], sems[b], add=True)

    def drain(b):
        pltpu.make_async_copy(ones_v, acc.at[rowv[b]], sems[b]).wait()

    for b in range(_DNB):
        unpack(b, b)
        fire(b)

    def group(g, carry):
        for b in range(_DNB):
            drain(b)
            unpack(_DNB * g + b, b)
            fire(b)
        return carry

    lax.fori_loop(1, NPK // _DNB, group, 0)
    for b in range(_DNB):
        drain(b)
    plsc.subcore_barrier()
    pltpu.sync_copy(acc.at[pl.ds(s * RPT, RPT)], stage)
    pltpu.sync_copy(stage, out_hbm.at[c, pl.ds(s * RPT, RPT)])


@functools.cache
def _deg_call():
    return pl.kernel(
        _deg_body,
        out_type=jax.ShapeDtypeStruct((NC, NPAD), jnp.float32),
        mesh=_sc_mesh(),
        scratch_types=[
            pltpu.VMEM((EPW,), jnp.int32),
            [pltpu.VMEM((PKW,), jnp.int32) for _ in range(_DNB)],
            pltpu.VMEM((PKW,), jnp.float32),
            pltpu.VMEM((RPT,), jnp.float32),
            pltpu.VMEM_SHARED((NPAD,), jnp.float32),
        ] + [pltpu.SemaphoreType.DMA] * _DNB,
        compiler_params=pltpu.CompilerParams(use_tc_tiling_on_sc=False),
    )


_NB = 5
_K = 3


def _spmm_body(d, ck, halves, nb, g_hbm, pk_hbm, out_hbm,
               pkvm, colv, rowv, rows, acc, isem, semg, sems):
    c = lax.axis_index("c")
    s = lax.axis_index("s")
    wid = s * NC + c
    nchunk = EPW // ck
    nz = RPT // ck
    win = EPW // halves

    def zrow(r, carry):
        for k in range(d // 16):
            rows[0][r, pl.ds(k * 16, 16)] = jnp.zeros((16,), jnp.float32)
        return carry

    lax.fori_loop(0, ck, zrow, 0)
    zcp = [pltpu.async_copy(rows[0],
                            acc.at[pl.ds(s * RPT + i * ck, ck)], isem)
           for i in range(nz)]
    pltpu.sync_copy(pk_hbm.at[pl.ds(wid * EPW, win)], pkvm)
    for z in zcp:
        z.wait()
    plsc.subcore_barrier()

    def unpack(j, b):
        off = lax.rem(j * ck, win) if halves > 1 else j * ck
        for k in range(ck // 16):
            v = pkvm[pl.ds(off + k * 16, 16)]
            colv[b][pl.ds(k * 16, 16)] = jnp.bitwise_and(v, 0xFFFF)
            rowv[b][pl.ds(k * 16, 16)] = lax.shift_right_logical(v, 16)

    def fire_g(j, b):
        return pltpu.async_copy(g_hbm.at[colv[b]], rows[b], semg[b])

    def wait_g(j, b):
        pltpu.make_async_copy(g_hbm.at[colv[b]], rows[b], semg[b]).wait()

    def fire_s(j, b):
        return pltpu.async_copy(rows[b], acc.at[rowv[b]], sems[b], add=True)

    def wait_s(j, b):
        pltpu.make_async_copy(rows[b], acc.at[rowv[b]], sems[b]).wait()

    for b in range(nb):
        unpack(b, b)
        fire_g(b, b)
    for j in range(nb - _K):
        wait_g(j, j)
        fire_s(j, j)

    def group(g, carry):
        for b in range(nb):
            j = nb * g + b
            wait_s(j - nb, b)
            unpack(j, b)
            fire_g(j, b)
            jk = j - _K
            bk = (b - _K) % nb
            wait_g(jk, bk)
            fire_s(jk, bk)
        return carry

    ngrp = nchunk // nb
    if halves == 1:
        lax.fori_loop(1, ngrp, group, 0)
    else:
        lax.fori_loop(1, ngrp // 2, group, 0)
        pltpu.sync_copy(pk_hbm.at[pl.ds(wid * EPW + win, win)], pkvm)
        lax.fori_loop(ngrp // 2, ngrp, group, 0)
    for j in range(nchunk - _K, nchunk):
        wait_g(j, j % nb)
        fire_s(j, j % nb)
    for b in range(nb):
        wait_s(nchunk - nb + b, b)
    plsc.subcore_barrier()

    def rd_pair(i):
        return (acc.at[pl.ds(s * RPT + i * ck, ck)], rows[i % nb],
                semg[i % nb])

    def wr_pair(i):
        return (rows[i % nb], out_hbm.at[c, pl.ds(s * RPT + i * ck, ck)],
                sems[i % nb])

    for i in range(nz):
        if i >= nb:
            pltpu.make_async_copy(*wr_pair(i - nb)).wait()
        pltpu.async_copy(*rd_pair(i))
        if i >= 1:
            pltpu.make_async_copy(*rd_pair(i - 1)).wait()
            pltpu.async_copy(*wr_pair(i - 1))
    pltpu.make_async_copy(*rd_pair(nz - 1)).wait()
    pltpu.async_copy(*wr_pair(nz - 1))
    for i in range(max(0, nz - nb), nz):
        pltpu.make_async_copy(*wr_pair(i)).wait()


@functools.cache
def _make_spmm(d, ck, halves, nb):
    return pl.kernel(
        functools.partial(_spmm_body, d, ck, halves, nb),
        out_type=jax.ShapeDtypeStruct((NC, NPAD, d), jnp.float32),
        mesh=_sc_mesh(),
        scratch_types=[
            pltpu.VMEM((EPW // halves,), jnp.int32),
            [pltpu.VMEM((ck,), jnp.int32) for _ in range(nb)],
            [pltpu.VMEM((ck,), jnp.int32) for _ in range(nb)],
            [pltpu.VMEM((ck, d), jnp.float32) for _ in range(nb)],
            pltpu.VMEM_SHARED((NPAD, d), jnp.float32),
            pltpu.SemaphoreType.DMA,
            [pltpu.SemaphoreType.DMA for _ in range(nb)],
            [pltpu.SemaphoreType.DMA for _ in range(nb)],
        ],
        compiler_params=pltpu.CompilerParams(use_tc_tiling_on_sc=False),
    )


_RB = 2048
_GRID = (N + _RB - 1) // _RB


def _pack_body(ei_ref, pk_ref):
    pk_ref[...] = jnp.left_shift(ei_ref[0], 16) | ei_ref[1]


def _pack(ei):
    return pl.pallas_call(
        _pack_body,
        out_shape=jax.ShapeDtypeStruct((E,), jnp.int32),
    )(ei)


def _dense0_body(deg_ref, x_ref, w_ref, dinv_ref, g0_ref):
    dinv = lax.rsqrt(deg_ref[0] + deg_ref[1] + 1.0)
    dinv_ref[...] = dinv
    g0_ref[...] = dinv[:, None] * jnp.dot(x_ref[...], w_ref[...],
                                          preferred_element_type=jnp.float32)


def _dense0(degp, x, w0):
    return pl.pallas_call(
        _dense0_body,
        grid=(_GRID,),
        in_specs=[
            pl.BlockSpec((NC, _RB), lambda i: (0, i)),
            pl.BlockSpec((_RB, D_IN), lambda i: (i, 0)),
            pl.BlockSpec((D_IN, D_HID), lambda i: (0, 0)),
        ],
        out_specs=[
            pl.BlockSpec((_RB,), lambda i: (i,)),
            pl.BlockSpec((_RB, D_HID), lambda i: (i, 0)),
        ],
        out_shape=[
            jax.ShapeDtypeStruct((N,), jnp.float32),
            jax.ShapeDtypeStruct((N, D_HID), jnp.float32),
        ],
    )(degp, x, w0)


def _dense1_body(p_ref, g0_ref, dinv_ref, m1_ref, m2_ref, w_ref, g1_ref):
    dinv = dinv_ref[...][:, None]
    a = dinv * (p_ref[0] + p_ref[1] + g0_ref[...])
    h = jnp.maximum(a, 0.0) * m1_ref[...] * m2_ref[...]
    g1_ref[...] = dinv * jnp.dot(h, w_ref[...],
                                 preferred_element_type=jnp.float32)


def _dense1(acc0, g0, dinv, m1, m2, w1):
    return pl.pallas_call(
        _dense1_body,
        grid=(_GRID,),
        in_specs=[
            pl.BlockSpec((NC, _RB, D_HID), lambda i: (0, i, 0)),
            pl.BlockSpec((_RB, D_HID), lambda i: (i, 0)),
            pl.BlockSpec((_RB,), lambda i: (i,)),
            pl.BlockSpec((_RB, D_HID), lambda i: (i, 0)),
            pl.BlockSpec((_RB, D_HID), lambda i: (i, 0)),
            pl.BlockSpec((D_HID, D_OUT), lambda i: (0, 0)),
        ],
        out_specs=pl.BlockSpec((_RB, D_OUT), lambda i: (i, 0)),
        out_shape=jax.ShapeDtypeStruct((N, D_OUT), jnp.float32),
    )(acc0, g0, dinv, m1, m2, w1)


def _dense2_body(q_ref, g1_ref, dinv_ref, out_ref):
    out_ref[...] = dinv_ref[...][:, None] * (q_ref[0] + q_ref[1]
                                             + g1_ref[...])


def _dense2(acc1, g1, dinv):
    return pl.pallas_call(
        _dense2_body,
        grid=(_GRID,),
        in_specs=[
            pl.BlockSpec((NC, _RB, D_OUT), lambda i: (0, i, 0)),
            pl.BlockSpec((_RB, D_OUT), lambda i: (i, 0)),
            pl.BlockSpec((_RB,), lambda i: (i,)),
        ],
        out_specs=pl.BlockSpec((_RB, D_OUT), lambda i: (i, 0)),
        out_shape=jax.ShapeDtypeStruct((N, D_OUT), jnp.float32),
    )(acc1, g1, dinv)


def kernel(x, edge_index, W0, W1, adj_mask1_train, adj_mask2_fixed):
    ei = edge_index.astype(jnp.int32)
    npad = EP - E
    ar = jnp.arange(npad, dtype=jnp.int32)
    pad = jnp.left_shift(DUMMY + ar % (NPAD - N), 16) | (ar % N)

    packed = jnp.concatenate([_pack(ei), pad])
    degp = _deg_call()(packed)
    dinv, g0 = _dense0(degp, x, W0)
    acc0 = _make_spmm(D_HID, 64, 2, 5)(g0, packed)
    g1 = _dense1(acc0, g0, dinv,
                 adj_mask1_train, adj_mask2_fixed, W1)
    acc1 = _make_spmm(D_OUT, 128, 1, 5)(g1, packed)
    return _dense2(acc1, g1, dinv)


# speedup vs baseline: 1.0713x; 1.0215x over previous
import functools

import jax
import jax.numpy as jnp
from jax import lax
from jax.experimental import pallas as pl
from jax.experimental.pallas import tpu as pltpu
from jax.experimental.pallas import tpu_sc as plsc

N = 10000
E = 320000
D_IN = 128
D_HID = 128
D_OUT = 64

NC = 2
NS = 16
NW = NC * NS

PKW = 128
EPW = 10240
EP = NW * EPW
NPK = EPW // PKW
NPAD = 10240
RPT = NPAD // NS
DUMMY = N


def _sc_mesh():
    return plsc.VectorSubcoreMesh(core_axis_name="c", subcore_axis_name="s",
                                  num_cores=NC, num_subcores=NS)


_DNB = 8


def _deg_body(pk_hbm, out_hbm, pkvm, rowv, ones_v, stage, acc, *sems):
    c = lax.axis_index("c")
    s = lax.axis_index("s")
    wid = s * NC + c
    for k in range(PKW // 16):
        ones_v[pl.ds(k * 16, 16)] = jnp.ones((16,), jnp.float32)

    def zloop(i, carry):
        stage[pl.ds(i * 16, 16)] = jnp.zeros((16,), jnp.float32)
        return carry

    lax.fori_loop(0, RPT // 16, zloop, 0)
    pltpu.sync_copy(pk_hbm.at[pl.ds(wid * EPW, EPW)], pkvm)
    pltpu.sync_copy(stage, acc.at[pl.ds(s * RPT, RPT)])
    plsc.subcore_barrier()

    def unpack(m, b):
        for k in range(PKW // 16):
            v = pkvm[pl.ds(m * PKW + k * 16, 16)]
            rowv[b][pl.ds(k * 16, 16)] = lax.shift_right_logical(v, 16)

    def fire(b):
        return pltpu.async_copy(ones_v, acc.at[rowv[b]], sems[b], add=True)

    def drain(b):
        pltpu.make_async_copy(ones_v, acc.at[rowv[b]], sems[b]).wait()

    for b in range(_DNB):
        unpack(b, b)
        fire(b)

    def group(g, carry):
        for b in range(_DNB):
            drain(b)
            unpack(_DNB * g + b, b)
            fire(b)
        return carry

    lax.fori_loop(1, NPK // _DNB, group, 0)
    for b in range(_DNB):
        drain(b)
    plsc.subcore_barrier()
    pltpu.sync_copy(acc.at[pl.ds(s * RPT, RPT)], stage)
    pltpu.sync_copy(stage, out_hbm.at[c, pl.ds(s * RPT, RPT)])


@functools.cache
def _deg_call():
    return pl.kernel(
        _deg_body,
        out_type=jax.ShapeDtypeStruct((NC, NPAD), jnp.float32),
        mesh=_sc_mesh(),
        scratch_types=[
            pltpu.VMEM((EPW,), jnp.int32),
            [pltpu.VMEM((PKW,), jnp.int32) for _ in range(_DNB)],
            pltpu.VMEM((PKW,), jnp.float32),
            pltpu.VMEM((RPT,), jnp.float32),
            pltpu.VMEM_SHARED((NPAD,), jnp.float32),
        ] + [pltpu.SemaphoreType.DMA] * _DNB,
        compiler_params=pltpu.CompilerParams(use_tc_tiling_on_sc=False),
    )


_NB = 5
_K = 4


def _spmm_body(d, ck, halves, nb, g_hbm, pk_hbm, out_hbm,
               pkvm, colv, rowv, rows, acc, isem, semg, sems):
    c = lax.axis_index("c")
    s = lax.axis_index("s")
    wid = s * NC + c
    nchunk = EPW // ck
    nz = RPT // ck
    win = EPW // halves

    def zrow(r, carry):
        for k in range(d // 16):
            rows[0][r, pl.ds(k * 16, 16)] = jnp.zeros((16,), jnp.float32)
        return carry

    lax.fori_loop(0, ck, zrow, 0)
    zcp = [pltpu.async_copy(rows[0],
                            acc.at[pl.ds(s * RPT + i * ck, ck)], isem)
           for i in range(nz)]
    pltpu.sync_copy(pk_hbm.at[pl.ds(wid * EPW, win)], pkvm)
    for z in zcp:
        z.wait()
    plsc.subcore_barrier()

    def unpack(j, b):
        off = lax.rem(j * ck, win) if halves > 1 else j * ck
        for k in range(ck // 16):
            v = pkvm[pl.ds(off + k * 16, 16)]
            colv[b][pl.ds(k * 16, 16)] = jnp.bitwise_and(v, 0xFFFF)
            rowv[b][pl.ds(k * 16, 16)] = lax.shift_right_logical(v, 16)

    def fire_g(j, b):
        return pltpu.async_copy(g_hbm.at[colv[b]], rows[b], semg[b])

    def wait_g(j, b):
        pltpu.make_async_copy(g_hbm.at[colv[b]], rows[b], semg[b]).wait()

    def fire_s(j, b):
        return pltpu.async_copy(rows[b], acc.at[rowv[b]], sems[b], add=True)

    def wait_s(j, b):
        pltpu.make_async_copy(rows[b], acc.at[rowv[b]], sems[b]).wait()

    for b in range(nb):
        unpack(b, b)
        fire_g(b, b)
    for j in range(nb - _K):
        wait_g(j, j)
        fire_s(j, j)

    def group(g, carry):
        for b in range(nb):
            j = nb * g + b
            wait_s(j - nb, b)
            unpack(j, b)
            fire_g(j, b)
            jk = j - _K
            bk = (b - _K) % nb
            wait_g(jk, bk)
            fire_s(jk, bk)
        return carry

    ngrp = nchunk // nb
    if halves == 1:
        lax.fori_loop(1, ngrp, group, 0)
    else:
        lax.fori_loop(1, ngrp // 2, group, 0)
        pltpu.sync_copy(pk_hbm.at[pl.ds(wid * EPW + win, win)], pkvm)
        lax.fori_loop(ngrp // 2, ngrp, group, 0)
    for j in range(nchunk - _K, nchunk):
        wait_g(j, j % nb)
        fire_s(j, j % nb)
    for b in range(nb):
        wait_s(nchunk - nb + b, b)
    plsc.subcore_barrier()

    def rd_pair(i):
        return (acc.at[pl.ds(s * RPT + i * ck, ck)], rows[i % nb],
                semg[i % nb])

    def wr_pair(i):
        return (rows[i % nb], out_hbm.at[c, pl.ds(s * RPT + i * ck, ck)],
                sems[i % nb])

    for i in range(nz):
        if i >= nb:
            pltpu.make_async_copy(*wr_pair(i - nb)).wait()
        pltpu.async_copy(*rd_pair(i))
        if i >= 1:
            pltpu.make_async_copy(*rd_pair(i - 1)).wait()
            pltpu.async_copy(*wr_pair(i - 1))
    pltpu.make_async_copy(*rd_pair(nz - 1)).wait()
    pltpu.async_copy(*wr_pair(nz - 1))
    for i in range(max(0, nz - nb), nz):
        pltpu.make_async_copy(*wr_pair(i)).wait()


@functools.cache
def _make_spmm(d, ck, halves, nb):
    return pl.kernel(
        functools.partial(_spmm_body, d, ck, halves, nb),
        out_type=jax.ShapeDtypeStruct((NC, NPAD, d), jnp.float32),
        mesh=_sc_mesh(),
        scratch_types=[
            pltpu.VMEM((EPW // halves,), jnp.int32),
            [pltpu.VMEM((ck,), jnp.int32) for _ in range(nb)],
            [pltpu.VMEM((ck,), jnp.int32) for _ in range(nb)],
            [pltpu.VMEM((ck, d), jnp.float32) for _ in range(nb)],
            pltpu.VMEM_SHARED((NPAD, d), jnp.float32),
            pltpu.SemaphoreType.DMA,
            [pltpu.SemaphoreType.DMA for _ in range(nb)],
            [pltpu.SemaphoreType.DMA for _ in range(nb)],
        ],
        compiler_params=pltpu.CompilerParams(use_tc_tiling_on_sc=False),
    )


_RB = 2048
_GRID = (N + _RB - 1) // _RB


def _pack_body(ei_ref, pk_ref):
    pk_ref[...] = jnp.left_shift(ei_ref[0], 16) | ei_ref[1]


def _pack(ei):
    return pl.pallas_call(
        _pack_body,
        out_shape=jax.ShapeDtypeStruct((E,), jnp.int32),
    )(ei)


def _dense0_body(deg_ref, x_ref, w_ref, dinv_ref, g0_ref):
    dinv = lax.rsqrt(deg_ref[0] + deg_ref[1] + 1.0)
    dinv_ref[...] = dinv
    g0_ref[...] = dinv[:, None] * jnp.dot(x_ref[...], w_ref[...],
                                          preferred_element_type=jnp.float32)


def _dense0(degp, x, w0):
    return pl.pallas_call(
        _dense0_body,
        grid=(_GRID,),
        in_specs=[
            pl.BlockSpec((NC, _RB), lambda i: (0, i)),
            pl.BlockSpec((_RB, D_IN), lambda i: (i, 0)),
            pl.BlockSpec((D_IN, D_HID), lambda i: (0, 0)),
        ],
        out_specs=[
            pl.BlockSpec((_RB,), lambda i: (i,)),
            pl.BlockSpec((_RB, D_HID), lambda i: (i, 0)),
        ],
        out_shape=[
            jax.ShapeDtypeStruct((N,), jnp.float32),
            jax.ShapeDtypeStruct((N, D_HID), jnp.float32),
        ],
    )(degp, x, w0)


def _dense1_body(p_ref, g0_ref, dinv_ref, m1_ref, m2_ref, w_ref, g1_ref):
    dinv = dinv_ref[...][:, None]
    a = dinv * (p_ref[0] + p_ref[1] + g0_ref[...])
    h = jnp.maximum(a, 0.0) * m1_ref[...] * m2_ref[...]
    g1_ref[...] = dinv * jnp.dot(h, w_ref[...],
                                 preferred_element_type=jnp.float32)


def _dense1(acc0, g0, dinv, m1, m2, w1):
    return pl.pallas_call(
        _dense1_body,
        grid=(_GRID,),
        in_specs=[
            pl.BlockSpec((NC, _RB, D_HID), lambda i: (0, i, 0)),
            pl.BlockSpec((_RB, D_HID), lambda i: (i, 0)),
            pl.BlockSpec((_RB,), lambda i: (i,)),
            pl.BlockSpec((_RB, D_HID), lambda i: (i, 0)),
            pl.BlockSpec((_RB, D_HID), lambda i: (i, 0)),
            pl.BlockSpec((D_HID, D_OUT), lambda i: (0, 0)),
        ],
        out_specs=pl.BlockSpec((_RB, D_OUT), lambda i: (i, 0)),
        out_shape=jax.ShapeDtypeStruct((N, D_OUT), jnp.float32),
    )(acc0, g0, dinv, m1, m2, w1)


def _dense2_body(q_ref, g1_ref, dinv_ref, out_ref):
    out_ref[...] = dinv_ref[...][:, None] * (q_ref[0] + q_ref[1]
                                             + g1_ref[...])


def _dense2(acc1, g1, dinv):
    return pl.pallas_call(
        _dense2_body,
        grid=(_GRID,),
        in_specs=[
            pl.BlockSpec((NC, _RB, D_OUT), lambda i: (0, i, 0)),
            pl.BlockSpec((_RB, D_OUT), lambda i: (i, 0)),
            pl.BlockSpec((_RB,), lambda i: (i,)),
        ],
        out_specs=pl.BlockSpec((_RB, D_OUT), lambda i: (i, 0)),
        out_shape=jax.ShapeDtypeStruct((N, D_OUT), jnp.float32),
    )(acc1, g1, dinv)


def kernel(x, edge_index, W0, W1, adj_mask1_train, adj_mask2_fixed):
    ei = edge_index.astype(jnp.int32)
    npad = EP - E
    ar = jnp.arange(npad, dtype=jnp.int32)
    pad = jnp.left_shift(DUMMY + ar % (NPAD - N), 16) | (ar % N)

    packed = jnp.concatenate([_pack(ei), pad])
    degp = _deg_call()(packed)
    dinv, g0 = _dense0(degp, x, W0)
    acc0 = _make_spmm(D_HID, 64, 2, 5)(g0, packed)
    g1 = _dense1(acc0, g0, dinv,
                 adj_mask1_train, adj_mask2_fixed, W1)
    acc1 = _make_spmm(D_OUT, 128, 1, 5)(g1, packed)
    return _dense2(acc1, g1, dinv)
